# baseline (speedup 1.0000x reference)
import functools

import jax
import jax.numpy as jnp
from jax import lax
from jax.experimental import pallas as pl
from jax.experimental.pallas import tpu as pltpu
from jax.experimental.pallas import tpu_sc as plsc

MASK_RATIO = 0.4
MASK_VALUE = 0.0

_SC_LANES = 16


def _mask_kernel(noise_ref, mask_ref, mult_ref, *, num_remove):
    n = noise_ref[0]
    S = n.shape[-1]
    k2 = pltpu.bitcast(n, jnp.int32) * 2
    j_idx = lax.broadcasted_iota(jnp.int32, (1, S, S), 1)
    i_idx = lax.broadcasted_iota(jnp.int32, (1, S, S), 2)
    tri = (j_idx > i_idx).astype(jnp.int32)
    bj = k2[:, :, None] + tri
    greater = bj > k2[:, None, :]
    cnt = jnp.count_nonzero(greater, axis=1).astype(jnp.int32)
    remove = cnt < num_remove
    mask_ref[0] = remove.astype(jnp.float32)
    mult_ref[0] = jnp.where(remove, jnp.float32(MASK_VALUE), jnp.float32(1.0))


def _compute_mask(noise, num_remove):
    batch, channels, seq = noise.shape
    return pl.pallas_call(
        functools.partial(_mask_kernel, num_remove=num_remove),
        grid=(batch,),
        in_specs=[pl.BlockSpec((1, channels, seq), lambda b: (b, 0, 0))],
        out_specs=[
            pl.BlockSpec((1, channels, seq), lambda b: (b, 0, 0)),
            pl.BlockSpec((1, channels, seq), lambda b: (b, 0, 0)),
        ],
        out_shape=[
            jax.ShapeDtypeStruct((batch, channels, seq), jnp.float32),
            jax.ShapeDtypeStruct((batch, channels, seq), jnp.float32),
        ],
    )(noise)


def _sc_fill(patch_flat, mult):
    batch, channels, sf = patch_flat.shape
    seq = mult.shape[-1]
    feat = sf // seq
    mesh = plsc.VectorSubcoreMesh(core_axis_name="core",
                                  subcore_axis_name="subcore")

    @pl.kernel(
        out_type=jax.ShapeDtypeStruct((batch, channels, sf), jnp.float32),
        mesh=mesh,
    )
    def fill_kernel(mult_hbm, patch_hbm, out_hbm):
        def body(m_vmem, x_vmem, o_vmem):
            @pl.loop(0, seq)
            def _row(s):
                m = m_vmem[0, 0, pl.ds(s, 1)][0]

                @pl.loop(0, feat, step=_SC_LANES)
                def _vec(f):
                    sl = (0, 0, pl.ds(s * feat + f, _SC_LANES))
                    o_vmem[sl] = x_vmem[sl] * m

        pltpu.emit_pipeline(
            body,
            grid=(batch, channels),
            in_specs=[
                pl.BlockSpec((1, 1, seq), lambda b, c: (b, c, 0)),
                pl.BlockSpec((1, 1, sf), lambda b, c: (b, c, 0)),
            ],
            out_specs=[pl.BlockSpec((1, 1, sf), lambda b, c: (b, c, 0))],
            core_axis_name=("core", "subcore"),
            dimension_semantics=(pltpu.PARALLEL, pltpu.PARALLEL),
        )(mult_hbm, patch_hbm, out_hbm)

    return fill_kernel(mult, patch_flat)


def kernel(patch_input, noise):
    batch, channels, seq, feat = patch_input.shape
    len_keep = int(seq * (1 - MASK_RATIO))
    num_remove = seq - len_keep

    mask, mult = _compute_mask(noise, num_remove)
    out = _sc_fill(patch_input.reshape(batch, channels, seq * feat), mult)
    return out.reshape(batch, channels, seq, feat), mask.astype(bool)


# speedup vs baseline: 1.0379x; 1.0379x over previous
import functools

import jax
import jax.numpy as jnp
from jax import lax
from jax.experimental import pallas as pl
from jax.experimental.pallas import tpu as pltpu

MASK_RATIO = 0.4
MASK_VALUE = 0.0

Q = 4


def _rank_counts(n):
    S = n.shape[-1]
    k2 = pltpu.bitcast(n, jnp.int32) * 2
    j_idx = lax.broadcasted_iota(jnp.int32, (1, S, S), 1)
    i_idx = lax.broadcasted_iota(jnp.int32, (1, S, S), 2)
    tri = (j_idx > i_idx).astype(jnp.int32)
    bj = k2[:, :, None] + tri
    greater = bj > k2[:, None, :]
    return jnp.count_nonzero(greater, axis=1).astype(jnp.int32)


def _fused_kernel(*refs, num_remove):
    noise_refs = refs[:Q]
    patch_refs = refs[Q:2 * Q]
    out_refs = refs[2 * Q:3 * Q]
    mask_refs = refs[3 * Q:4 * Q]
    for q in range(Q):
        n = noise_refs[q][0]
        cnt = _rank_counts(n)
        mask_refs[q][0] = (cnt < num_remove).astype(jnp.float32)
        x = patch_refs[q][0]
        out_refs[q][0] = jnp.where(cnt[:, :, None] < num_remove,
                                   jnp.float32(MASK_VALUE), x)


def kernel(patch_input, noise):
    batch, channels, seq, feat = patch_input.shape
    len_keep = int(seq * (1 - MASK_RATIO))
    num_remove = seq - len_keep
    steps = batch // Q

    def nmap(q):
        return lambda b: (b + steps * q, 0, 0)

    def pmap(q):
        return lambda b: (b + steps * q, 0, 0, 0)

    outs = pl.pallas_call(
        functools.partial(_fused_kernel, num_remove=num_remove),
        grid=(steps,),
        in_specs=(
            [pl.BlockSpec((1, channels, seq), nmap(q)) for q in range(Q)]
            + [pl.BlockSpec((1, channels, seq, feat), pmap(q)) for q in range(Q)]
        ),
        out_specs=(
            [pl.BlockSpec((1, channels, seq, feat), pmap(q)) for q in range(Q)]
            + [pl.BlockSpec((1, channels, seq), nmap(q)) for q in range(Q)]
        ),
        out_shape=(
            [jax.ShapeDtypeStruct((batch, channels, seq, feat),
                                  patch_input.dtype)] * Q
            + [jax.ShapeDtypeStruct((batch, channels, seq), jnp.float32)] * Q
        ),
    )(*([noise] * Q + [patch_input] * Q))

    out = jnp.concatenate(
        [outs[q][steps * q:steps * (q + 1)] for q in range(Q)], axis=0)
    mask = jnp.concatenate(
        [outs[Q + q][steps * q:steps * (q + 1)] for q in range(Q)], axis=0)
    return out, mask.astype(bool)


# speedup vs baseline: 1.4296x; 1.3775x over previous
import functools

import jax
import jax.numpy as jnp
from jax import lax
from jax.experimental import pallas as pl
from jax.experimental.pallas import tpu as pltpu
from jax.experimental.pallas import tpu_sc as plsc

MASK_RATIO = 0.4
MASK_VALUE = 0.0

_SC_LANES = 16


def _mask_kernel(noise_ref, mask_ref, mult_ref, *, num_remove):
    n = noise_ref[0]
    S = n.shape[-1]
    k2 = pltpu.bitcast(n, jnp.int32) * 2
    j_idx = lax.broadcasted_iota(jnp.int32, (1, S, S), 1)
    i_idx = lax.broadcasted_iota(jnp.int32, (1, S, S), 2)
    tri = (j_idx > i_idx).astype(jnp.int32)
    bj = k2[:, :, None] + tri
    greater = bj > k2[:, None, :]
    cnt = jnp.count_nonzero(greater, axis=1).astype(jnp.int32)
    remove = cnt < num_remove
    mask_ref[0] = remove.astype(jnp.float32)
    mult_ref[0] = jnp.where(remove, jnp.float32(MASK_VALUE), jnp.float32(1.0))


def _compute_mask(noise, num_remove):
    batch, channels, seq = noise.shape
    return pl.pallas_call(
        functools.partial(_mask_kernel, num_remove=num_remove),
        grid=(batch,),
        in_specs=[pl.BlockSpec((1, channels, seq), lambda b: (b, 0, 0))],
        out_specs=[
            pl.BlockSpec((1, channels, seq), lambda b: (b, 0, 0)),
            pl.BlockSpec((1, channels, seq), lambda b: (b, 0, 0)),
        ],
        out_shape=[
            jax.ShapeDtypeStruct((batch, channels, seq), jnp.float32),
            jax.ShapeDtypeStruct((batch, channels, seq), jnp.float32),
        ],
    )(noise)


def _sc_fill(patch_flat, mult):
    batch, channels, sf = patch_flat.shape
    seq = mult.shape[-1]
    feat = sf // seq
    mesh = plsc.VectorSubcoreMesh(core_axis_name="core",
                                  subcore_axis_name="subcore")

    @pl.kernel(
        out_type=jax.ShapeDtypeStruct((batch, channels, sf), jnp.float32),
        mesh=mesh,
    )
    def fill_kernel(mult_hbm, patch_hbm, out_hbm):
        def body(m_vmem, x_vmem, o_vmem):
            @plsc.parallel_loop(0, seq, unroll=4)
            def _row(s):
                m = m_vmem[0, 0, pl.ds(s, 1)][0]
                base = s * feat
                for f in range(0, feat, _SC_LANES):
                    sl = (0, 0, pl.ds(base + f, _SC_LANES))
                    o_vmem[sl] = x_vmem[sl] * m

        pltpu.emit_pipeline(
            body,
            grid=(batch, channels),
            in_specs=[
                pl.BlockSpec((1, 1, seq), lambda b, c: (b, c, 0)),
                pl.BlockSpec((1, 1, sf), lambda b, c: (b, c, 0)),
            ],
            out_specs=[pl.BlockSpec((1, 1, sf), lambda b, c: (b, c, 0))],
            core_axis_name=("core", "subcore"),
            dimension_semantics=(pltpu.PARALLEL, pltpu.PARALLEL),
        )(mult_hbm, patch_hbm, out_hbm)

    return fill_kernel(mult, patch_flat)


def kernel(patch_input, noise):
    batch, channels, seq, feat = patch_input.shape
    len_keep = int(seq * (1 - MASK_RATIO))
    num_remove = seq - len_keep

    mask, mult = _compute_mask(noise, num_remove)
    out = _sc_fill(patch_input.reshape(batch, channels, seq * feat), mult)
    return out.reshape(batch, channels, seq, feat), mask.astype(bool)
